# baseline (speedup 1.0000x reference)
import functools

import jax
import jax.numpy as jnp
import numpy as np
from jax import lax
from jax.experimental import pallas as pl
from jax.experimental.pallas import tpu as pltpu
from jax.experimental.pallas import tpu_sc as plsc

NUM_NODES = 2048
BATCH = 4
N = NUM_NODES * BATCH
E = 131072
NF = 128
EMB = 16
HEADS = 4
GC = 32
D = HEADS * GC
ED = 16
PROJ = 4
OUT = 128
NGAT = 4

NW = 32
EPT = E // NW
GRP = EPT // 128
NB = 8
BLK = N // NB


def _make_edge_call():
    mesh = plsc.VectorSubcoreMesh(core_axis_name="c", subcore_axis_name="s")

    @functools.partial(
        pl.kernel,
        out_type=[
            jax.ShapeDtypeStruct((2, N, D), jnp.float32),
            jax.ShapeDtypeStruct((2, N, 16), jnp.float32),
        ],
        mesh=mesh,
        scratch_types=[
            pltpu.VMEM((GRP, 128), jnp.int32),
            pltpu.VMEM((GRP, 128), jnp.int32),
            [pltpu.VMEM((128, D), jnp.float32)] * 2,
            [pltpu.VMEM((128, 16), jnp.float32)] * 2,
            [pltpu.VMEM((128, 16), jnp.float32)] * 2,
            [pltpu.VMEM((16, 128), jnp.float32)] * 2,
            [pltpu.VMEM((128, 16), jnp.float32)] * 2,
            pltpu.VMEM_SHARED((N, D), jnp.float32),
            pltpu.VMEM_SHARED((N, 16), jnp.float32),
            [pltpu.SemaphoreType.DMA] * 12,
        ],
        compiler_params=pltpu.CompilerParams(use_tc_tiling_on_sc=False),
    )
    def edge_call(xw_hbm, tabs_hbm, tabd_hbm, ale_hbm, src_hbm, dst_hbm,
                  omsg_hbm, oden_hbm,
                  src_v, dst_v, gx, gs, gad, gae, exb, accm, accd, sems):
        cid = lax.axis_index("c")
        sid = lax.axis_index("s")
        wid = sid * 2 + cid
        rps = N // 16

        zv = jnp.zeros((16,), jnp.float32)

        def zrow(i, c):
            for v in range(D // 16):
                gx[0][i, 16 * v:16 * (v + 1)] = zv
            exb[0][i, :] = zv
            return c

        lax.fori_loop(0, 128, zrow, 0)
        for rep in range(rps // 128):
            pltpu.sync_copy(gx[0], accm.at[pl.ds(sid * rps + rep * 128, 128)])
            pltpu.sync_copy(exb[0], accd.at[pl.ds(sid * rps + rep * 128, 128)])
        plsc.subcore_barrier()

        pltpu.sync_copy(src_hbm.at[wid], src_v)
        pltpu.sync_copy(dst_hbm.at[wid], dst_v)

        def issue_gather(g, b):
            base8 = wid * (EPT // 8) + g * 16
            pltpu.async_copy(xw_hbm.at[src_v.at[g]], gx[b], sems[6 * b + 0])
            pltpu.async_copy(tabs_hbm.at[src_v.at[g]], gs[b], sems[6 * b + 1])
            pltpu.async_copy(tabd_hbm.at[dst_v.at[g]], gad[b], sems[6 * b + 2])
            pltpu.async_copy(ale_hbm.at[pl.ds(base8, 16)], gae[b], sems[6 * b + 3])

        def wait_gather(g, b):
            base8 = wid * (EPT // 8) + g * 16
            pltpu.make_async_copy(xw_hbm.at[src_v.at[g]], gx[b], sems[6 * b + 0]).wait()
            pltpu.make_async_copy(tabs_hbm.at[src_v.at[g]], gs[b], sems[6 * b + 1]).wait()
            pltpu.make_async_copy(tabd_hbm.at[dst_v.at[g]], gad[b], sems[6 * b + 2]).wait()
            pltpu.make_async_copy(ale_hbm.at[pl.ds(base8, 16)], gae[b], sems[6 * b + 3]).wait()

        def compute(b):
            gxb, gsb, gadb, gaeb, exbb = gx[b], gs[b], gad[b], gae[b], exb[b]

            def row(r, c2):
                for j in range(8):
                    e = 8 * r + j
                    a = gsb[e, :] + gadb[e, :] + gaeb[r, 16 * j:16 * j + 16]
                    a = jnp.where(a > 0.0, a, a * 0.2)
                    exv = jnp.exp(a)
                    exbb[e, :] = exv
                    for h in range(HEADS):
                        bv = jnp.full((16,), exv[h], jnp.float32)
                        gxb[e, 32 * h:32 * h + 16] = gxb[e, 32 * h:32 * h + 16] * bv
                        gxb[e, 32 * h + 16:32 * h + 32] = gxb[e, 32 * h + 16:32 * h + 32] * bv
                return c2

            lax.fori_loop(0, 16, row, 0, unroll=4)

        def issue_scatter(g, b):
            pltpu.async_copy(gx[b], accm.at[dst_v.at[g]], sems[6 * b + 4], add=True)
            pltpu.async_copy(exb[b], accd.at[dst_v.at[g]], sems[6 * b + 5], add=True)

        def wait_scatter(g, b):
            pltpu.make_async_copy(gx[b], accm.at[dst_v.at[g]], sems[6 * b + 4]).wait()
            pltpu.make_async_copy(exb[b], accd.at[dst_v.at[g]], sems[6 * b + 5]).wait()

        npairs = GRP // 2
        issue_gather(0, 0)

        def pair(p, c):
            g0 = 2 * p
            g1 = 2 * p + 1

            @pl.when(p > 0)
            def _():
                wait_scatter(g1, 1)

            issue_gather(g1, 1)
            wait_gather(g0, 0)
            compute(0)
            issue_scatter(g0, 0)
            wait_gather(g1, 1)
            compute(1)
            issue_scatter(g1, 1)
            wait_scatter(g0, 0)

            @pl.when(p < npairs - 1)
            def _():
                issue_gather(g0 + 2, 0)

            return c

        lax.fori_loop(0, npairs, pair, 0)
        wait_scatter(GRP - 1, 1)
        plsc.subcore_barrier()
        for rep in range(rps // 128):
            r0 = sid * rps + rep * 128
            pltpu.sync_copy(accm.at[pl.ds(r0, 128)], omsg_hbm.at[cid, pl.ds(r0, 128)])
            pltpu.sync_copy(accd.at[pl.ds(r0, 128)], oden_hbm.at[cid, pl.ds(r0, 128)])

    return edge_call


_edge_call = _make_edge_call()


def _dense0_body(x_ref, fcw_ref, fcb_ref, w_ref, asd_ref, ps_ref, pd_ref,
                 xw_ref, tabs_ref, tabd_ref):
    h0 = x_ref[...] @ fcw_ref[...] + fcb_ref[...]
    xw = h0 @ w_ref[...]
    sal = xw @ asd_ref[...]
    xw_ref[...] = xw
    tabs_ref[...] = sal @ ps_ref[...]
    tabd_ref[...] = sal @ pd_ref[...]


def _dense0(x_cat, fcw, fcb, w, asd, ps, pd):
    return pl.pallas_call(
        _dense0_body,
        grid=(NB,),
        in_specs=[
            pl.BlockSpec((BLK, NF + EMB), lambda i: (i, 0)),
            pl.BlockSpec((NF + EMB, 3 * EMB), lambda i: (0, 0)),
            pl.BlockSpec((1, 3 * EMB), lambda i: (0, 0)),
            pl.BlockSpec((3 * EMB, D), lambda i: (0, 0)),
            pl.BlockSpec((D, 2 * HEADS), lambda i: (0, 0)),
            pl.BlockSpec((2 * HEADS, 16), lambda i: (0, 0)),
            pl.BlockSpec((2 * HEADS, 16), lambda i: (0, 0)),
        ],
        out_specs=[
            pl.BlockSpec((BLK, D), lambda i: (i, 0)),
            pl.BlockSpec((BLK, 16), lambda i: (i, 0)),
            pl.BlockSpec((BLK, 16), lambda i: (i, 0)),
        ],
        out_shape=[
            jax.ShapeDtypeStruct((N, D), jnp.float32),
            jax.ShapeDtypeStruct((N, 16), jnp.float32),
            jax.ShapeDtypeStruct((N, 16), jnp.float32),
        ],
    )(x_cat, fcw, fcb, w, asd, ps, pd)


def _mid_body(pm_ref, pd_ref, b_ref, qm_ref, w_ref, asd_ref, ps_ref, psd_ref,
              xw_ref, tabs_ref, tabd_ref):
    den = (pd_ref[0] + pd_ref[1]) @ qm_ref[...]
    h = (pm_ref[0] + pm_ref[1]) / (den + 1e-16) + b_ref[...]
    h = jnp.where(h > 0.0, h, 0.01 * h)
    xw = h @ w_ref[...]
    sal = xw @ asd_ref[...]
    xw_ref[...] = xw
    tabs_ref[...] = sal @ ps_ref[...]
    tabd_ref[...] = sal @ psd_ref[...]


def _mid(pmsg, pden, bias, qmat, w, asd, ps, pd):
    return pl.pallas_call(
        _mid_body,
        grid=(NB,),
        in_specs=[
            pl.BlockSpec((2, BLK, D), lambda i: (0, i, 0)),
            pl.BlockSpec((2, BLK, 16), lambda i: (0, i, 0)),
            pl.BlockSpec((1, D), lambda i: (0, 0)),
            pl.BlockSpec((16, D), lambda i: (0, 0)),
            pl.BlockSpec((D, D), lambda i: (0, 0)),
            pl.BlockSpec((D, 2 * HEADS), lambda i: (0, 0)),
            pl.BlockSpec((2 * HEADS, 16), lambda i: (0, 0)),
            pl.BlockSpec((2 * HEADS, 16), lambda i: (0, 0)),
        ],
        out_specs=[
            pl.BlockSpec((BLK, D), lambda i: (i, 0)),
            pl.BlockSpec((BLK, 16), lambda i: (i, 0)),
            pl.BlockSpec((BLK, 16), lambda i: (i, 0)),
        ],
        out_shape=[
            jax.ShapeDtypeStruct((N, D), jnp.float32),
            jax.ShapeDtypeStruct((N, 16), jnp.float32),
            jax.ShapeDtypeStruct((N, 16), jnp.float32),
        ],
    )(pmsg, pden, bias, qmat, w, asd, ps, pd)


def _post_body(pm_ref, pd_ref, b_ref, qm_ref, wq_ref, bq_ref, wk_ref, bk_ref,
               wv_ref, bv_ref, q_ref, k_ref, v_ref):
    den = (pd_ref[0] + pd_ref[1]) @ qm_ref[...]
    h = (pm_ref[0] + pm_ref[1]) / (den + 1e-16) + b_ref[...]
    h = jnp.where(h > 0.0, h, 0.01 * h)
    q_ref[...] = h @ wq_ref[...] + bq_ref[...]
    k_ref[...] = h @ wk_ref[...] + bk_ref[...]
    v_ref[...] = h @ wv_ref[...] + bv_ref[...]


def _post(pmsg, pden, bias, qmat, wq, bq, wk, bk, wv, bv):
    wspec = pl.BlockSpec((D, D), lambda i: (0, 0))
    bspec = pl.BlockSpec((1, D), lambda i: (0, 0))
    nspec = pl.BlockSpec((BLK, D), lambda i: (i, 0))
    return pl.pallas_call(
        _post_body,
        grid=(NB,),
        in_specs=[
            pl.BlockSpec((2, BLK, D), lambda i: (0, i, 0)),
            pl.BlockSpec((2, BLK, 16), lambda i: (0, i, 0)),
            bspec,
            pl.BlockSpec((16, D), lambda i: (0, 0)),
            wspec, bspec, wspec, bspec, wspec, bspec,
        ],
        out_specs=[nspec, nspec, nspec],
        out_shape=[jax.ShapeDtypeStruct((N, D), jnp.float32)] * 3,
    )(pmsg, pden, bias, qmat, wq, bq, wk, bk, wv, bv)


def _ale_body(ea_ref, bd_ref, *outs):
    ea = ea_ref[...]
    for l in range(NGAT):
        outs[l][...] = ea @ bd_ref[...][:, 128 * l:128 * (l + 1)]


def _ale(ea_pack, bd_all):
    eb = E // 8 // 32
    return pl.pallas_call(
        _ale_body,
        grid=(32,),
        in_specs=[
            pl.BlockSpec((eb, 128), lambda i: (i, 0)),
            pl.BlockSpec((128, 128 * NGAT), lambda i: (0, 0)),
        ],
        out_specs=[pl.BlockSpec((eb, 128), lambda i: (i, 0))] * NGAT,
        out_shape=[jax.ShapeDtypeStruct((E // 8, 128), jnp.float32)] * NGAT,
    )(ea_pack, bd_all)


QB = 256
_SCALE = 1.0 / np.sqrt(GC)


def _attn_body(q_ref, k_ref, v_ref, wop_ref, cbp_ref, z_ref):
    q = q_ref[0]
    k = k_ref[0]
    v = v_ref[0]
    outs = []
    for h in range(HEADS):
        sl = slice(GC * h, GC * (h + 1))
        s = lax.dot_general(q[:, sl] * _SCALE, k[:, sl],
                            (((1,), (1,)), ((), ())),
                            preferred_element_type=jnp.float32)
        m = jnp.max(s, axis=1, keepdims=True)
        p = jnp.exp(s - m)
        att = p / jnp.sum(p, axis=1, keepdims=True)
        outs.append(att @ v[:, sl])
    o = jnp.concatenate(outs, axis=1)
    z_ref[0] = o @ wop_ref[...] + cbp_ref[...]


def _attn(q, k, v, wop, cbp):
    return pl.pallas_call(
        _attn_body,
        grid=(BATCH, NUM_NODES // QB),
        in_specs=[
            pl.BlockSpec((1, QB, D), lambda b, i: (b, i, 0)),
            pl.BlockSpec((1, NUM_NODES, D), lambda b, i: (b, 0, 0)),
            pl.BlockSpec((1, NUM_NODES, D), lambda b, i: (b, 0, 0)),
            pl.BlockSpec((D, PROJ), lambda b, i: (0, 0)),
            pl.BlockSpec((1, PROJ), lambda b, i: (0, 0)),
        ],
        out_specs=pl.BlockSpec((1, QB, PROJ), lambda b, i: (b, i, 0)),
        out_shape=jax.ShapeDtypeStruct((BATCH, NUM_NODES, PROJ), jnp.float32),
    )(q, k, v, wop, cbp)


def _fin_body(z_ref, w_ref, b_ref, o_ref):
    @pl.when(pl.program_id(0) == 0)
    def _():
        o_ref[...] = jnp.broadcast_to(b_ref[...], (BATCH, OUT))

    o_ref[...] += z_ref[...] @ w_ref[...]


def _fin(z2, out_w, out_b):
    kb = 1024
    return pl.pallas_call(
        _fin_body,
        grid=(NUM_NODES * PROJ // kb),
        in_specs=[
            pl.BlockSpec((BATCH, kb), lambda i: (0, i)),
            pl.BlockSpec((kb, OUT), lambda i: (i, 0)),
            pl.BlockSpec((1, OUT), lambda i: (0, 0)),
        ],
        out_specs=pl.BlockSpec((BATCH, OUT), lambda i: (0, 0)),
        out_shape=jax.ShapeDtypeStruct((BATCH, OUT), jnp.float32),
    )(z2, out_w, out_b)


def _make_asd(a_s, a_d):
    hidx = jnp.arange(D) // GC
    sel = (hidx[:, None] == jnp.arange(HEADS)[None, :]).astype(jnp.float32)
    return jnp.concatenate(
        [a_s.reshape(D)[:, None] * sel, a_d.reshape(D)[:, None] * sel], axis=1)


def _make_be(we, ae):
    return jnp.pad(
        jnp.einsum("dhc,hc->dh", we.reshape(ED, HEADS, GC), ae.reshape(HEADS, GC)),
        ((0, 0), (0, 16 - HEADS)))


def kernel(x, edge_index, edge_attr, batch, emb, fc_W, fc_b, g0_W, g0_as,
           g0_ad, g0_We, g0_ae, g0_b, gW, gAs, gAd, gWe, gAe, gB, Wq, bq,
           Wk, bk, Wv, bv, Wo, bo, proj_W, proj_b, out_W, out_b):
    f32 = jnp.float32
    x_cat = jnp.concatenate([x, jnp.tile(emb, (BATCH, 1))], axis=1)

    ps = (jnp.arange(2 * HEADS)[:, None] ==
          jnp.arange(16)[None, :]).astype(f32)
    pd = (jnp.arange(2 * HEADS)[:, None] ==
          (jnp.arange(16)[None, :] + HEADS)).astype(f32)
    qmat = (jnp.arange(16)[:, None] ==
            (jnp.arange(D)[None, :] // GC)).astype(f32)

    asd = [_make_asd(g0_as, g0_ad)] + [
        _make_asd(gAs[i], gAd[i]) for i in range(NGAT - 1)]
    eye8 = jnp.eye(8, dtype=f32)
    bd_all = jnp.concatenate(
        [jnp.kron(eye8, _make_be(g0_We, g0_ae))] +
        [jnp.kron(eye8, _make_be(gWe[i], gAe[i])) for i in range(NGAT - 1)],
        axis=1)
    ws = [g0_W] + [gW[i] for i in range(NGAT - 1)]
    biases = [g0_b] + [gB[i] for i in range(NGAT - 1)]

    ale = _ale(edge_attr.reshape(E // 8, 8 * ED), bd_all)
    src3 = edge_index[0].reshape(NW, GRP, 128)
    dst3 = edge_index[1].reshape(NW, GRP, 128)

    xw, tabs, tabd = _dense0(x_cat, fc_W, fc_b.reshape(1, -1), ws[0],
                             asd[0], ps, pd)
    for l in range(NGAT):
        pmsg, pden = _edge_call(xw, tabs, tabd, ale[l], src3, dst3)
        if l < NGAT - 1:
            xw, tabs, tabd = _mid(pmsg, pden, biases[l].reshape(1, -1), qmat,
                                  ws[l + 1], asd[l + 1], ps, pd)
        else:
            q, k, v = _post(pmsg, pden, biases[l].reshape(1, -1), qmat,
                            Wq, bq.reshape(1, -1), Wk, bk.reshape(1, -1),
                            Wv, bv.reshape(1, -1))

    wop = Wo @ proj_W
    cbp = (bo @ proj_W + proj_b).reshape(1, -1)
    z = _attn(q.reshape(BATCH, NUM_NODES, D),
              k.reshape(BATCH, NUM_NODES, D),
              v.reshape(BATCH, NUM_NODES, D), wop, cbp)
    z2 = z.reshape(BATCH, NUM_NODES * PROJ)
    return _fin(z2, out_W, out_b.reshape(1, -1))


# speedup vs baseline: 1.0258x; 1.0258x over previous
import functools

import jax
import jax.numpy as jnp
import numpy as np
from jax import lax
from jax.experimental import pallas as pl
from jax.experimental.pallas import tpu as pltpu
from jax.experimental.pallas import tpu_sc as plsc

NUM_NODES = 2048
BATCH = 4
N = NUM_NODES * BATCH
E = 131072
NF = 128
EMB = 16
HEADS = 4
GC = 32
D = HEADS * GC
ED = 16
PROJ = 4
OUT = 128
NGAT = 4

NW = 32
EPT = E // NW
GRP = EPT // 128
NB = 8
BLK = N // NB


def _make_edge_call():
    mesh = plsc.VectorSubcoreMesh(core_axis_name="c", subcore_axis_name="s")

    @functools.partial(
        pl.kernel,
        out_type=[
            jax.ShapeDtypeStruct((2, N, D), jnp.float32),
            jax.ShapeDtypeStruct((2, N, 16), jnp.float32),
        ],
        mesh=mesh,
        scratch_types=[
            pltpu.VMEM((GRP, 128), jnp.int32),
            pltpu.VMEM((GRP, 128), jnp.int32),
            [pltpu.VMEM((128, D), jnp.float32)] * 2,
            [pltpu.VMEM((128, 16), jnp.float32)] * 2,
            [pltpu.VMEM((128, 16), jnp.float32)] * 2,
            [pltpu.VMEM((16, 128), jnp.float32)] * 2,
            [pltpu.VMEM((128, 16), jnp.float32)] * 2,
            pltpu.VMEM_SHARED((N, D), jnp.float32),
            pltpu.VMEM_SHARED((N, 16), jnp.float32),
            [pltpu.SemaphoreType.DMA] * 12,
        ],
        compiler_params=pltpu.CompilerParams(use_tc_tiling_on_sc=False),
    )
    def edge_call(xw_hbm, tabs_hbm, tabd_hbm, ale_hbm, src_hbm, dst_hbm,
                  omsg_hbm, oden_hbm,
                  src_v, dst_v, gx, gs, gad, gae, exb, accm, accd, sems):
        cid = lax.axis_index("c")
        sid = lax.axis_index("s")
        wid = sid * 2 + cid
        rps = N // 16

        zv = jnp.zeros((16,), jnp.float32)

        def zrow(i, c):
            for v in range(D // 16):
                gx[0][i, 16 * v:16 * (v + 1)] = zv
            exb[0][i, :] = zv
            return c

        lax.fori_loop(0, 128, zrow, 0)
        for rep in range(rps // 128):
            pltpu.sync_copy(gx[0], accm.at[pl.ds(sid * rps + rep * 128, 128)])
            pltpu.sync_copy(exb[0], accd.at[pl.ds(sid * rps + rep * 128, 128)])
        plsc.subcore_barrier()

        pltpu.sync_copy(src_hbm.at[wid], src_v)
        pltpu.sync_copy(dst_hbm.at[wid], dst_v)

        def issue_gather(g, b):
            base8 = wid * (EPT // 8) + g * 16
            pltpu.async_copy(xw_hbm.at[src_v.at[g]], gx[b], sems[6 * b + 0])
            pltpu.async_copy(tabs_hbm.at[src_v.at[g]], gs[b], sems[6 * b + 1])
            pltpu.async_copy(tabd_hbm.at[dst_v.at[g]], gad[b], sems[6 * b + 2])
            pltpu.async_copy(ale_hbm.at[pl.ds(base8, 16)], gae[b], sems[6 * b + 3])

        def wait_gather(g, b):
            base8 = wid * (EPT // 8) + g * 16
            pltpu.make_async_copy(xw_hbm.at[src_v.at[g]], gx[b], sems[6 * b + 0]).wait()
            pltpu.make_async_copy(tabs_hbm.at[src_v.at[g]], gs[b], sems[6 * b + 1]).wait()
            pltpu.make_async_copy(tabd_hbm.at[dst_v.at[g]], gad[b], sems[6 * b + 2]).wait()
            pltpu.make_async_copy(ale_hbm.at[pl.ds(base8, 16)], gae[b], sems[6 * b + 3]).wait()

        def compute(b):
            gxb, gsb, gadb, gaeb, exbb = gx[b], gs[b], gad[b], gae[b], exb[b]

            def row(r, c2):
                for j in range(8):
                    e = 8 * r + j
                    a = gsb[e, :] + gadb[e, :] + gaeb[r, 16 * j:16 * j + 16]
                    a = jnp.where(a > 0.0, a, a * 0.2)
                    exv = jnp.exp(a)
                    exbb[e, :] = exv
                    for h in range(HEADS):
                        bv = jnp.full((16,), exv[h], jnp.float32)
                        gxb[e, 32 * h:32 * h + 16] = gxb[e, 32 * h:32 * h + 16] * bv
                        gxb[e, 32 * h + 16:32 * h + 32] = gxb[e, 32 * h + 16:32 * h + 32] * bv
                return c2

            lax.fori_loop(0, 16, row, 0, unroll=2)

        def issue_scatter(g, b):
            pltpu.async_copy(gx[b], accm.at[dst_v.at[g]], sems[6 * b + 4], add=True)
            pltpu.async_copy(exb[b], accd.at[dst_v.at[g]], sems[6 * b + 5], add=True)

        def wait_scatter(g, b):
            pltpu.make_async_copy(gx[b], accm.at[dst_v.at[g]], sems[6 * b + 4]).wait()
            pltpu.make_async_copy(exb[b], accd.at[dst_v.at[g]], sems[6 * b + 5]).wait()

        npairs = GRP // 2
        issue_gather(0, 0)

        def pair(p, c):
            g0 = 2 * p
            g1 = 2 * p + 1

            @pl.when(p > 0)
            def _():
                wait_scatter(g1, 1)

            issue_gather(g1, 1)
            wait_gather(g0, 0)
            compute(0)
            issue_scatter(g0, 0)
            wait_gather(g1, 1)
            compute(1)
            issue_scatter(g1, 1)
            wait_scatter(g0, 0)

            @pl.when(p < npairs - 1)
            def _():
                issue_gather(g0 + 2, 0)

            return c

        lax.fori_loop(0, npairs, pair, 0)
        wait_scatter(GRP - 1, 1)
        plsc.subcore_barrier()
        for rep in range(rps // 128):
            r0 = sid * rps + rep * 128
            pltpu.sync_copy(accm.at[pl.ds(r0, 128)], omsg_hbm.at[cid, pl.ds(r0, 128)])
            pltpu.sync_copy(accd.at[pl.ds(r0, 128)], oden_hbm.at[cid, pl.ds(r0, 128)])

    return edge_call


_edge_call = _make_edge_call()


def _dense0_body(x_ref, fcw_ref, fcb_ref, w_ref, asd_ref, ps_ref, pd_ref,
                 xw_ref, tabs_ref, tabd_ref):
    h0 = x_ref[...] @ fcw_ref[...] + fcb_ref[...]
    xw = h0 @ w_ref[...]
    sal = xw @ asd_ref[...]
    xw_ref[...] = xw
    tabs_ref[...] = sal @ ps_ref[...]
    tabd_ref[...] = sal @ pd_ref[...]


def _dense0(x_cat, fcw, fcb, w, asd, ps, pd):
    return pl.pallas_call(
        _dense0_body,
        grid=(NB,),
        in_specs=[
            pl.BlockSpec((BLK, NF + EMB), lambda i: (i, 0)),
            pl.BlockSpec((NF + EMB, 3 * EMB), lambda i: (0, 0)),
            pl.BlockSpec((1, 3 * EMB), lambda i: (0, 0)),
            pl.BlockSpec((3 * EMB, D), lambda i: (0, 0)),
            pl.BlockSpec((D, 2 * HEADS), lambda i: (0, 0)),
            pl.BlockSpec((2 * HEADS, 16), lambda i: (0, 0)),
            pl.BlockSpec((2 * HEADS, 16), lambda i: (0, 0)),
        ],
        out_specs=[
            pl.BlockSpec((BLK, D), lambda i: (i, 0)),
            pl.BlockSpec((BLK, 16), lambda i: (i, 0)),
            pl.BlockSpec((BLK, 16), lambda i: (i, 0)),
        ],
        out_shape=[
            jax.ShapeDtypeStruct((N, D), jnp.float32),
            jax.ShapeDtypeStruct((N, 16), jnp.float32),
            jax.ShapeDtypeStruct((N, 16), jnp.float32),
        ],
    )(x_cat, fcw, fcb, w, asd, ps, pd)


def _mid_body(pm_ref, pd_ref, b_ref, qm_ref, w_ref, asd_ref, ps_ref, psd_ref,
              xw_ref, tabs_ref, tabd_ref):
    den = (pd_ref[0] + pd_ref[1]) @ qm_ref[...]
    h = (pm_ref[0] + pm_ref[1]) / (den + 1e-16) + b_ref[...]
    h = jnp.where(h > 0.0, h, 0.01 * h)
    xw = h @ w_ref[...]
    sal = xw @ asd_ref[...]
    xw_ref[...] = xw
    tabs_ref[...] = sal @ ps_ref[...]
    tabd_ref[...] = sal @ psd_ref[...]


def _mid(pmsg, pden, bias, qmat, w, asd, ps, pd):
    return pl.pallas_call(
        _mid_body,
        grid=(NB,),
        in_specs=[
            pl.BlockSpec((2, BLK, D), lambda i: (0, i, 0)),
            pl.BlockSpec((2, BLK, 16), lambda i: (0, i, 0)),
            pl.BlockSpec((1, D), lambda i: (0, 0)),
            pl.BlockSpec((16, D), lambda i: (0, 0)),
            pl.BlockSpec((D, D), lambda i: (0, 0)),
            pl.BlockSpec((D, 2 * HEADS), lambda i: (0, 0)),
            pl.BlockSpec((2 * HEADS, 16), lambda i: (0, 0)),
            pl.BlockSpec((2 * HEADS, 16), lambda i: (0, 0)),
        ],
        out_specs=[
            pl.BlockSpec((BLK, D), lambda i: (i, 0)),
            pl.BlockSpec((BLK, 16), lambda i: (i, 0)),
            pl.BlockSpec((BLK, 16), lambda i: (i, 0)),
        ],
        out_shape=[
            jax.ShapeDtypeStruct((N, D), jnp.float32),
            jax.ShapeDtypeStruct((N, 16), jnp.float32),
            jax.ShapeDtypeStruct((N, 16), jnp.float32),
        ],
    )(pmsg, pden, bias, qmat, w, asd, ps, pd)


def _post_body(pm_ref, pd_ref, b_ref, qm_ref, wq_ref, bq_ref, wk_ref, bk_ref,
               wv_ref, bv_ref, q_ref, k_ref, v_ref):
    den = (pd_ref[0] + pd_ref[1]) @ qm_ref[...]
    h = (pm_ref[0] + pm_ref[1]) / (den + 1e-16) + b_ref[...]
    h = jnp.where(h > 0.0, h, 0.01 * h)
    q_ref[...] = h @ wq_ref[...] + bq_ref[...]
    k_ref[...] = h @ wk_ref[...] + bk_ref[...]
    v_ref[...] = h @ wv_ref[...] + bv_ref[...]


def _post(pmsg, pden, bias, qmat, wq, bq, wk, bk, wv, bv):
    wspec = pl.BlockSpec((D, D), lambda i: (0, 0))
    bspec = pl.BlockSpec((1, D), lambda i: (0, 0))
    nspec = pl.BlockSpec((BLK, D), lambda i: (i, 0))
    return pl.pallas_call(
        _post_body,
        grid=(NB,),
        in_specs=[
            pl.BlockSpec((2, BLK, D), lambda i: (0, i, 0)),
            pl.BlockSpec((2, BLK, 16), lambda i: (0, i, 0)),
            bspec,
            pl.BlockSpec((16, D), lambda i: (0, 0)),
            wspec, bspec, wspec, bspec, wspec, bspec,
        ],
        out_specs=[nspec, nspec, nspec],
        out_shape=[jax.ShapeDtypeStruct((N, D), jnp.float32)] * 3,
    )(pmsg, pden, bias, qmat, wq, bq, wk, bk, wv, bv)


def _ale_body(ea_ref, bd_ref, *outs):
    ea = ea_ref[...]
    for l in range(NGAT):
        outs[l][...] = ea @ bd_ref[...][:, 128 * l:128 * (l + 1)]


def _ale(ea_pack, bd_all):
    eb = E // 8 // 32
    return pl.pallas_call(
        _ale_body,
        grid=(32,),
        in_specs=[
            pl.BlockSpec((eb, 128), lambda i: (i, 0)),
            pl.BlockSpec((128, 128 * NGAT), lambda i: (0, 0)),
        ],
        out_specs=[pl.BlockSpec((eb, 128), lambda i: (i, 0))] * NGAT,
        out_shape=[jax.ShapeDtypeStruct((E // 8, 128), jnp.float32)] * NGAT,
    )(ea_pack, bd_all)


QB = 256
_SCALE = 1.0 / np.sqrt(GC)


def _attn_body(q_ref, k_ref, v_ref, wop_ref, cbp_ref, z_ref):
    bf16 = jnp.bfloat16
    q = (q_ref[0] * _SCALE).astype(bf16)
    k = k_ref[0].astype(bf16)
    v = v_ref[0].astype(bf16)
    outs = []
    for h in range(HEADS):
        sl = slice(GC * h, GC * (h + 1))
        s = lax.dot_general(q[:, sl], k[:, sl],
                            (((1,), (1,)), ((), ())),
                            preferred_element_type=jnp.float32)
        m = jnp.max(s, axis=1, keepdims=True)
        p = jnp.exp(s - m)
        att = (p / jnp.sum(p, axis=1, keepdims=True)).astype(bf16)
        outs.append(lax.dot_general(att, v[:, sl], (((1,), (0,)), ((), ())),
                                    preferred_element_type=jnp.float32))
    o = jnp.concatenate(outs, axis=1)
    z_ref[0] = o @ wop_ref[...] + cbp_ref[...]


def _attn(q, k, v, wop, cbp):
    return pl.pallas_call(
        _attn_body,
        grid=(BATCH, NUM_NODES // QB),
        in_specs=[
            pl.BlockSpec((1, QB, D), lambda b, i: (b, i, 0)),
            pl.BlockSpec((1, NUM_NODES, D), lambda b, i: (b, 0, 0)),
            pl.BlockSpec((1, NUM_NODES, D), lambda b, i: (b, 0, 0)),
            pl.BlockSpec((D, PROJ), lambda b, i: (0, 0)),
            pl.BlockSpec((1, PROJ), lambda b, i: (0, 0)),
        ],
        out_specs=pl.BlockSpec((1, QB, PROJ), lambda b, i: (b, i, 0)),
        out_shape=jax.ShapeDtypeStruct((BATCH, NUM_NODES, PROJ), jnp.float32),
    )(q, k, v, wop, cbp)


def _fin_body(z_ref, w_ref, b_ref, o_ref):
    @pl.when(pl.program_id(0) == 0)
    def _():
        o_ref[...] = jnp.broadcast_to(b_ref[...], (BATCH, OUT))

    o_ref[...] += z_ref[...] @ w_ref[...]


def _fin(z2, out_w, out_b):
    kb = 1024
    return pl.pallas_call(
        _fin_body,
        grid=(NUM_NODES * PROJ // kb),
        in_specs=[
            pl.BlockSpec((BATCH, kb), lambda i: (0, i)),
            pl.BlockSpec((kb, OUT), lambda i: (i, 0)),
            pl.BlockSpec((1, OUT), lambda i: (0, 0)),
        ],
        out_specs=pl.BlockSpec((BATCH, OUT), lambda i: (0, 0)),
        out_shape=jax.ShapeDtypeStruct((BATCH, OUT), jnp.float32),
    )(z2, out_w, out_b)


def _make_asd(a_s, a_d):
    hidx = jnp.arange(D) // GC
    sel = (hidx[:, None] == jnp.arange(HEADS)[None, :]).astype(jnp.float32)
    return jnp.concatenate(
        [a_s.reshape(D)[:, None] * sel, a_d.reshape(D)[:, None] * sel], axis=1)


def _make_be(we, ae):
    return jnp.pad(
        jnp.einsum("dhc,hc->dh", we.reshape(ED, HEADS, GC), ae.reshape(HEADS, GC)),
        ((0, 0), (0, 16 - HEADS)))


def kernel(x, edge_index, edge_attr, batch, emb, fc_W, fc_b, g0_W, g0_as,
           g0_ad, g0_We, g0_ae, g0_b, gW, gAs, gAd, gWe, gAe, gB, Wq, bq,
           Wk, bk, Wv, bv, Wo, bo, proj_W, proj_b, out_W, out_b):
    f32 = jnp.float32
    x_cat = jnp.concatenate([x, jnp.tile(emb, (BATCH, 1))], axis=1)

    ps = (jnp.arange(2 * HEADS)[:, None] ==
          jnp.arange(16)[None, :]).astype(f32)
    pd = (jnp.arange(2 * HEADS)[:, None] ==
          (jnp.arange(16)[None, :] + HEADS)).astype(f32)
    qmat = (jnp.arange(16)[:, None] ==
            (jnp.arange(D)[None, :] // GC)).astype(f32)

    asd = [_make_asd(g0_as, g0_ad)] + [
        _make_asd(gAs[i], gAd[i]) for i in range(NGAT - 1)]
    eye8 = jnp.eye(8, dtype=f32)
    bd_all = jnp.concatenate(
        [jnp.kron(eye8, _make_be(g0_We, g0_ae))] +
        [jnp.kron(eye8, _make_be(gWe[i], gAe[i])) for i in range(NGAT - 1)],
        axis=1)
    ws = [g0_W] + [gW[i] for i in range(NGAT - 1)]
    biases = [g0_b] + [gB[i] for i in range(NGAT - 1)]

    ale = _ale(edge_attr.reshape(E // 8, 8 * ED), bd_all)
    src3 = edge_index[0].reshape(NW, GRP, 128)
    dst3 = edge_index[1].reshape(NW, GRP, 128)

    xw, tabs, tabd = _dense0(x_cat, fc_W, fc_b.reshape(1, -1), ws[0],
                             asd[0], ps, pd)
    for l in range(NGAT):
        pmsg, pden = _edge_call(xw, tabs, tabd, ale[l], src3, dst3)
        if l < NGAT - 1:
            xw, tabs, tabd = _mid(pmsg, pden, biases[l].reshape(1, -1), qmat,
                                  ws[l + 1], asd[l + 1], ps, pd)
        else:
            q, k, v = _post(pmsg, pden, biases[l].reshape(1, -1), qmat,
                            Wq, bq.reshape(1, -1), Wk, bk.reshape(1, -1),
                            Wv, bv.reshape(1, -1))

    wop = Wo @ proj_W
    cbp = (bo @ proj_W + proj_b).reshape(1, -1)
    z = _attn(q.reshape(BATCH, NUM_NODES, D),
              k.reshape(BATCH, NUM_NODES, D),
              v.reshape(BATCH, NUM_NODES, D), wop, cbp)
    z2 = z.reshape(BATCH, NUM_NODES * PROJ)
    return _fin(z2, out_W, out_b.reshape(1, -1))


# speedup vs baseline: 1.0928x; 1.0653x over previous
import functools

import jax
import jax.numpy as jnp
import numpy as np
from jax import lax
from jax.experimental import pallas as pl
from jax.experimental.pallas import tpu as pltpu
from jax.experimental.pallas import tpu_sc as plsc

NUM_NODES = 2048
BATCH = 4
N = NUM_NODES * BATCH
E = 131072
NF = 128
EMB = 16
HEADS = 4
GC = 32
D = HEADS * GC
ED = 16
PROJ = 4
OUT = 128
NGAT = 4

NW = 32
EPT = E // NW
GRP = EPT // 128
NB = 8
BLK = N // NB


def _make_edge_call():
    mesh = plsc.VectorSubcoreMesh(core_axis_name="c", subcore_axis_name="s")

    @functools.partial(
        pl.kernel,
        out_type=[
            jax.ShapeDtypeStruct((2, N, D), jnp.float32),
            jax.ShapeDtypeStruct((2, N, 16), jnp.float32),
        ],
        mesh=mesh,
        scratch_types=[
            pltpu.VMEM((GRP, 128), jnp.int32),
            pltpu.VMEM((GRP, 128), jnp.int32),
            [pltpu.VMEM((128, D), jnp.float32)] * 2,
            [pltpu.VMEM((128, 16), jnp.float32)] * 2,
            [pltpu.VMEM((128, 16), jnp.float32)] * 2,
            [pltpu.VMEM((16, 128), jnp.float32)] * 2,
            [pltpu.VMEM((128, 16), jnp.float32)] * 2,
            pltpu.VMEM_SHARED((N, D), jnp.float32),
            pltpu.VMEM_SHARED((N, 16), jnp.float32),
            [pltpu.SemaphoreType.DMA] * 12,
        ],
        compiler_params=pltpu.CompilerParams(use_tc_tiling_on_sc=False),
    )
    def edge_call(xw_hbm, tabs_hbm, tabd_hbm, ale_hbm, src_hbm, dst_hbm,
                  omsg_hbm, oden_hbm,
                  src_v, dst_v, gx, gs, gad, gae, exb, accm, accd, sems):
        cid = lax.axis_index("c")
        sid = lax.axis_index("s")
        wid = sid * 2 + cid
        rps = N // 16

        zv = jnp.zeros((16,), jnp.float32)

        def zrow(i, c):
            for v in range(D // 16):
                gx[0][i, 16 * v:16 * (v + 1)] = zv
            exb[0][i, :] = zv
            return c

        lax.fori_loop(0, 128, zrow, 0)
        for rep in range(rps // 128):
            pltpu.sync_copy(gx[0], accm.at[pl.ds(sid * rps + rep * 128, 128)])
            pltpu.sync_copy(exb[0], accd.at[pl.ds(sid * rps + rep * 128, 128)])
        plsc.subcore_barrier()

        pltpu.sync_copy(src_hbm.at[wid], src_v)
        pltpu.sync_copy(dst_hbm.at[wid], dst_v)

        def issue_gather(g, b):
            base8 = wid * (EPT // 8) + g * 16
            pltpu.async_copy(xw_hbm.at[src_v.at[g]], gx[b], sems[6 * b + 0])
            pltpu.async_copy(tabs_hbm.at[src_v.at[g]], gs[b], sems[6 * b + 1])
            pltpu.async_copy(tabd_hbm.at[dst_v.at[g]], gad[b], sems[6 * b + 2])
            pltpu.async_copy(ale_hbm.at[pl.ds(base8, 16)], gae[b], sems[6 * b + 3])

        def wait_gather(g, b):
            base8 = wid * (EPT // 8) + g * 16
            pltpu.make_async_copy(xw_hbm.at[src_v.at[g]], gx[b], sems[6 * b + 0]).wait()
            pltpu.make_async_copy(tabs_hbm.at[src_v.at[g]], gs[b], sems[6 * b + 1]).wait()
            pltpu.make_async_copy(tabd_hbm.at[dst_v.at[g]], gad[b], sems[6 * b + 2]).wait()
            pltpu.make_async_copy(ale_hbm.at[pl.ds(base8, 16)], gae[b], sems[6 * b + 3]).wait()

        def compute(b):
            gxb, gsb, gadb, gaeb, exbb = gx[b], gs[b], gad[b], gae[b], exb[b]

            def row(r, c2):
                for j in range(8):
                    e = 8 * r + j
                    a = gsb[e, :] + gadb[e, :] + gaeb[r, 16 * j:16 * j + 16]
                    a = jnp.where(a > 0.0, a, a * 0.2)
                    exv = jnp.exp(a)
                    exbb[e, :] = exv
                    for h in range(HEADS):
                        bv = jnp.full((16,), exv[h], jnp.float32)
                        gxb[e, 32 * h:32 * h + 16] = gxb[e, 32 * h:32 * h + 16] * bv
                        gxb[e, 32 * h + 16:32 * h + 32] = gxb[e, 32 * h + 16:32 * h + 32] * bv
                return c2

            lax.fori_loop(0, 16, row, 0, unroll=2)

        def issue_scatter(g, b):
            pltpu.async_copy(gx[b], accm.at[dst_v.at[g]], sems[6 * b + 4], add=True)
            pltpu.async_copy(exb[b], accd.at[dst_v.at[g]], sems[6 * b + 5], add=True)

        def wait_scatter(g, b):
            pltpu.make_async_copy(gx[b], accm.at[dst_v.at[g]], sems[6 * b + 4]).wait()
            pltpu.make_async_copy(exb[b], accd.at[dst_v.at[g]], sems[6 * b + 5]).wait()

        npairs = GRP // 2
        issue_gather(0, 0)

        def pair(p, c):
            g0 = 2 * p
            g1 = 2 * p + 1

            @pl.when(p > 0)
            def _():
                wait_scatter(g1, 1)

            issue_gather(g1, 1)
            wait_gather(g0, 0)
            compute(0)
            issue_scatter(g0, 0)
            wait_gather(g1, 1)
            compute(1)
            issue_scatter(g1, 1)
            wait_scatter(g0, 0)

            @pl.when(p < npairs - 1)
            def _():
                issue_gather(g0 + 2, 0)

            return c

        lax.fori_loop(0, npairs, pair, 0)
        wait_scatter(GRP - 1, 1)
        plsc.subcore_barrier()
        for rep in range(rps // 128):
            r0 = sid * rps + rep * 128
            pltpu.sync_copy(accm.at[pl.ds(r0, 128)], omsg_hbm.at[cid, pl.ds(r0, 128)])
            pltpu.sync_copy(accd.at[pl.ds(r0, 128)], oden_hbm.at[cid, pl.ds(r0, 128)])

    return edge_call


_edge_call = _make_edge_call()


def _dense0_body(x_ref, fcw_ref, fcb_ref, w_ref, asd_ref, ps_ref, pd_ref,
                 xw_ref, tabs_ref, tabd_ref):
    h0 = x_ref[...] @ fcw_ref[...] + fcb_ref[...]
    xw = h0 @ w_ref[...]
    sal = xw @ asd_ref[...]
    xw_ref[...] = xw
    tabs_ref[...] = sal @ ps_ref[...]
    tabd_ref[...] = sal @ pd_ref[...]


def _dense0(x_cat, fcw, fcb, w, asd, ps, pd):
    return pl.pallas_call(
        _dense0_body,
        grid=(NB,),
        in_specs=[
            pl.BlockSpec((BLK, NF + EMB), lambda i: (i, 0)),
            pl.BlockSpec((NF + EMB, 3 * EMB), lambda i: (0, 0)),
            pl.BlockSpec((1, 3 * EMB), lambda i: (0, 0)),
            pl.BlockSpec((3 * EMB, D), lambda i: (0, 0)),
            pl.BlockSpec((D, 2 * HEADS), lambda i: (0, 0)),
            pl.BlockSpec((2 * HEADS, 16), lambda i: (0, 0)),
            pl.BlockSpec((2 * HEADS, 16), lambda i: (0, 0)),
        ],
        out_specs=[
            pl.BlockSpec((BLK, D), lambda i: (i, 0)),
            pl.BlockSpec((BLK, 16), lambda i: (i, 0)),
            pl.BlockSpec((BLK, 16), lambda i: (i, 0)),
        ],
        out_shape=[
            jax.ShapeDtypeStruct((N, D), jnp.float32),
            jax.ShapeDtypeStruct((N, 16), jnp.float32),
            jax.ShapeDtypeStruct((N, 16), jnp.float32),
        ],
    )(x_cat, fcw, fcb, w, asd, ps, pd)


def _mid_body(pm_ref, pd_ref, b_ref, qm_ref, w_ref, asd_ref, ps_ref, psd_ref,
              xw_ref, tabs_ref, tabd_ref):
    den = (pd_ref[0] + pd_ref[1]) @ qm_ref[...]
    h = (pm_ref[0] + pm_ref[1]) / (den + 1e-16) + b_ref[...]
    h = jnp.where(h > 0.0, h, 0.01 * h)
    xw = h @ w_ref[...]
    sal = xw @ asd_ref[...]
    xw_ref[...] = xw
    tabs_ref[...] = sal @ ps_ref[...]
    tabd_ref[...] = sal @ psd_ref[...]


def _mid(pmsg, pden, bias, qmat, w, asd, ps, pd):
    return pl.pallas_call(
        _mid_body,
        grid=(NB,),
        in_specs=[
            pl.BlockSpec((2, BLK, D), lambda i: (0, i, 0)),
            pl.BlockSpec((2, BLK, 16), lambda i: (0, i, 0)),
            pl.BlockSpec((1, D), lambda i: (0, 0)),
            pl.BlockSpec((16, D), lambda i: (0, 0)),
            pl.BlockSpec((D, D), lambda i: (0, 0)),
            pl.BlockSpec((D, 2 * HEADS), lambda i: (0, 0)),
            pl.BlockSpec((2 * HEADS, 16), lambda i: (0, 0)),
            pl.BlockSpec((2 * HEADS, 16), lambda i: (0, 0)),
        ],
        out_specs=[
            pl.BlockSpec((BLK, D), lambda i: (i, 0)),
            pl.BlockSpec((BLK, 16), lambda i: (i, 0)),
            pl.BlockSpec((BLK, 16), lambda i: (i, 0)),
        ],
        out_shape=[
            jax.ShapeDtypeStruct((N, D), jnp.float32),
            jax.ShapeDtypeStruct((N, 16), jnp.float32),
            jax.ShapeDtypeStruct((N, 16), jnp.float32),
        ],
    )(pmsg, pden, bias, qmat, w, asd, ps, pd)


def _post_body(pm_ref, pd_ref, b_ref, qm_ref, wq_ref, bq_ref, wk_ref, bk_ref,
               wv_ref, bv_ref, q_ref, k_ref, v_ref):
    den = (pd_ref[0] + pd_ref[1]) @ qm_ref[...]
    h = (pm_ref[0] + pm_ref[1]) / (den + 1e-16) + b_ref[...]
    h = jnp.where(h > 0.0, h, 0.01 * h)
    q_ref[...] = h @ wq_ref[...] + bq_ref[...]
    k_ref[...] = h @ wk_ref[...] + bk_ref[...]
    v_ref[...] = h @ wv_ref[...] + bv_ref[...]


def _post(pmsg, pden, bias, qmat, wq, bq, wk, bk, wv, bv):
    wspec = pl.BlockSpec((D, D), lambda i: (0, 0))
    bspec = pl.BlockSpec((1, D), lambda i: (0, 0))
    nspec = pl.BlockSpec((BLK, D), lambda i: (i, 0))
    return pl.pallas_call(
        _post_body,
        grid=(NB,),
        in_specs=[
            pl.BlockSpec((2, BLK, D), lambda i: (0, i, 0)),
            pl.BlockSpec((2, BLK, 16), lambda i: (0, i, 0)),
            bspec,
            pl.BlockSpec((16, D), lambda i: (0, 0)),
            wspec, bspec, wspec, bspec, wspec, bspec,
        ],
        out_specs=[nspec, nspec, nspec],
        out_shape=[jax.ShapeDtypeStruct((N, D), jnp.float32)] * 3,
    )(pmsg, pden, bias, qmat, wq, bq, wk, bk, wv, bv)


def _ale_body(ea_ref, bd_ref, *outs):
    ea = ea_ref[...]
    for l in range(NGAT):
        outs[l][...] = ea @ bd_ref[...][:, 128 * l:128 * (l + 1)]


def _ale(ea_pack, bd_all):
    eb = E // 8 // 32
    return pl.pallas_call(
        _ale_body,
        grid=(32,),
        in_specs=[
            pl.BlockSpec((eb, 128), lambda i: (i, 0)),
            pl.BlockSpec((128, 128 * NGAT), lambda i: (0, 0)),
        ],
        out_specs=[pl.BlockSpec((eb, 128), lambda i: (i, 0))] * NGAT,
        out_shape=[jax.ShapeDtypeStruct((E // 8, 128), jnp.float32)] * NGAT,
    )(ea_pack, bd_all)


QB = 256
_SCALE = 1.0 / np.sqrt(GC)


def _attn_body(q_ref, k_ref, v_ref, wop_ref, cbp_ref, z_ref):
    bf16 = jnp.bfloat16
    q = (q_ref[0] * _SCALE).astype(bf16)
    k = k_ref[0].astype(bf16)
    v = v_ref[0].astype(bf16)
    outs = []
    for h in range(HEADS):
        sl = slice(GC * h, GC * (h + 1))
        s = lax.dot_general(q[:, sl], k[:, sl],
                            (((1,), (1,)), ((), ())),
                            preferred_element_type=jnp.float32)
        p = jnp.exp(s)
        rs = jnp.sum(p, axis=1, keepdims=True)
        oh = lax.dot_general(p.astype(bf16), v[:, sl],
                             (((1,), (0,)), ((), ())),
                             preferred_element_type=jnp.float32)
        outs.append(oh / rs)
    o = jnp.concatenate(outs, axis=1)
    z_ref[0] = o @ wop_ref[...] + cbp_ref[...]


def _attn(q, k, v, wop, cbp):
    return pl.pallas_call(
        _attn_body,
        grid=(BATCH, NUM_NODES // QB),
        in_specs=[
            pl.BlockSpec((1, QB, D), lambda b, i: (b, i, 0)),
            pl.BlockSpec((1, NUM_NODES, D), lambda b, i: (b, 0, 0)),
            pl.BlockSpec((1, NUM_NODES, D), lambda b, i: (b, 0, 0)),
            pl.BlockSpec((D, PROJ), lambda b, i: (0, 0)),
            pl.BlockSpec((1, PROJ), lambda b, i: (0, 0)),
        ],
        out_specs=pl.BlockSpec((1, QB, PROJ), lambda b, i: (b, i, 0)),
        out_shape=jax.ShapeDtypeStruct((BATCH, NUM_NODES, PROJ), jnp.float32),
    )(q, k, v, wop, cbp)


def _fin_body(z_ref, w_ref, b_ref, o_ref):
    @pl.when(pl.program_id(0) == 0)
    def _():
        o_ref[...] = jnp.broadcast_to(b_ref[...], (BATCH, OUT))

    o_ref[...] += z_ref[...] @ w_ref[...]


def _fin(z2, out_w, out_b):
    kb = 1024
    return pl.pallas_call(
        _fin_body,
        grid=(NUM_NODES * PROJ // kb),
        in_specs=[
            pl.BlockSpec((BATCH, kb), lambda i: (0, i)),
            pl.BlockSpec((kb, OUT), lambda i: (i, 0)),
            pl.BlockSpec((1, OUT), lambda i: (0, 0)),
        ],
        out_specs=pl.BlockSpec((BATCH, OUT), lambda i: (0, 0)),
        out_shape=jax.ShapeDtypeStruct((BATCH, OUT), jnp.float32),
    )(z2, out_w, out_b)


def _make_asd(a_s, a_d):
    hidx = jnp.arange(D) // GC
    sel = (hidx[:, None] == jnp.arange(HEADS)[None, :]).astype(jnp.float32)
    return jnp.concatenate(
        [a_s.reshape(D)[:, None] * sel, a_d.reshape(D)[:, None] * sel], axis=1)


def _make_be(we, ae):
    return jnp.pad(
        jnp.einsum("dhc,hc->dh", we.reshape(ED, HEADS, GC), ae.reshape(HEADS, GC)),
        ((0, 0), (0, 16 - HEADS)))


def kernel(x, edge_index, edge_attr, batch, emb, fc_W, fc_b, g0_W, g0_as,
           g0_ad, g0_We, g0_ae, g0_b, gW, gAs, gAd, gWe, gAe, gB, Wq, bq,
           Wk, bk, Wv, bv, Wo, bo, proj_W, proj_b, out_W, out_b):
    f32 = jnp.float32
    x_cat = jnp.concatenate([x, jnp.tile(emb, (BATCH, 1))], axis=1)

    ps = (jnp.arange(2 * HEADS)[:, None] ==
          jnp.arange(16)[None, :]).astype(f32)
    pd = (jnp.arange(2 * HEADS)[:, None] ==
          (jnp.arange(16)[None, :] + HEADS)).astype(f32)
    qmat = (jnp.arange(16)[:, None] ==
            (jnp.arange(D)[None, :] // GC)).astype(f32)

    asd = [_make_asd(g0_as, g0_ad)] + [
        _make_asd(gAs[i], gAd[i]) for i in range(NGAT - 1)]
    eye8 = jnp.eye(8, dtype=f32)
    bd_all = jnp.concatenate(
        [jnp.kron(eye8, _make_be(g0_We, g0_ae))] +
        [jnp.kron(eye8, _make_be(gWe[i], gAe[i])) for i in range(NGAT - 1)],
        axis=1)
    ws = [g0_W] + [gW[i] for i in range(NGAT - 1)]
    biases = [g0_b] + [gB[i] for i in range(NGAT - 1)]

    ale = _ale(edge_attr.reshape(E // 8, 8 * ED), bd_all)
    src3 = edge_index[0].reshape(NW, GRP, 128)
    dst3 = edge_index[1].reshape(NW, GRP, 128)

    xw, tabs, tabd = _dense0(x_cat, fc_W, fc_b.reshape(1, -1), ws[0],
                             asd[0], ps, pd)
    for l in range(NGAT):
        pmsg, pden = _edge_call(xw, tabs, tabd, ale[l], src3, dst3)
        if l < NGAT - 1:
            xw, tabs, tabd = _mid(pmsg, pden, biases[l].reshape(1, -1), qmat,
                                  ws[l + 1], asd[l + 1], ps, pd)
        else:
            q, k, v = _post(pmsg, pden, biases[l].reshape(1, -1), qmat,
                            Wq, bq.reshape(1, -1), Wk, bk.reshape(1, -1),
                            Wv, bv.reshape(1, -1))

    wop = Wo @ proj_W
    cbp = (bo @ proj_W + proj_b).reshape(1, -1)
    z = _attn(q.reshape(BATCH, NUM_NODES, D),
              k.reshape(BATCH, NUM_NODES, D),
              v.reshape(BATCH, NUM_NODES, D), wop, cbp)
    z2 = z.reshape(BATCH, NUM_NODES * PROJ)
    return _fin(z2, out_W, out_b.reshape(1, -1))


# speedup vs baseline: 1.1216x; 1.0263x over previous
import functools

import jax
import jax.numpy as jnp
import numpy as np
from jax import lax
from jax.experimental import pallas as pl
from jax.experimental.pallas import tpu as pltpu
from jax.experimental.pallas import tpu_sc as plsc

NUM_NODES = 2048
BATCH = 4
N = NUM_NODES * BATCH
E = 131072
NF = 128
EMB = 16
HEADS = 4
GC = 32
D = HEADS * GC
ED = 16
PROJ = 4
OUT = 128
NGAT = 4

NW = 32
EPT = E // NW
GRP = EPT // 128
NB = 8
BLK = N // NB


def _make_edge_call():
    mesh = plsc.VectorSubcoreMesh(core_axis_name="c", subcore_axis_name="s")

    @functools.partial(
        pl.kernel,
        out_type=[
            jax.ShapeDtypeStruct((2, N, D), jnp.float32),
            jax.ShapeDtypeStruct((2, N, 16), jnp.float32),
        ],
        mesh=mesh,
        scratch_types=[
            pltpu.VMEM((GRP, 128), jnp.int32),
            pltpu.VMEM((GRP, 128), jnp.int32),
            [pltpu.VMEM((128, D), jnp.float32)] * 2,
            [pltpu.VMEM((128, 16), jnp.float32)] * 2,
            [pltpu.VMEM((128, 16), jnp.float32)] * 2,
            [pltpu.VMEM((16, 128), jnp.float32)] * 2,
            [pltpu.VMEM((128, 16), jnp.float32)] * 2,
            pltpu.VMEM_SHARED((N, D), jnp.float32),
            pltpu.VMEM_SHARED((N, 16), jnp.float32),
            [pltpu.SemaphoreType.DMA] * 12,
        ],
        compiler_params=pltpu.CompilerParams(use_tc_tiling_on_sc=False),
    )
    def edge_call(xw_hbm, tabs_hbm, tabd_hbm, ale_hbm, src_hbm, dst_hbm,
                  omsg_hbm, oden_hbm,
                  src_v, dst_v, gx, gs, gad, gae, exb, accm, accd, sems):
        cid = lax.axis_index("c")
        sid = lax.axis_index("s")
        wid = sid * 2 + cid
        rps = N // 16

        zv = jnp.zeros((16,), jnp.float32)

        def zrow(i, c):
            for v in range(D // 16):
                gx[0][i, 16 * v:16 * (v + 1)] = zv
            exb[0][i, :] = zv
            return c

        lax.fori_loop(0, 128, zrow, 0)
        for rep in range(rps // 128):
            pltpu.sync_copy(gx[0], accm.at[pl.ds(sid * rps + rep * 128, 128)])
            pltpu.sync_copy(exb[0], accd.at[pl.ds(sid * rps + rep * 128, 128)])
        plsc.subcore_barrier()

        pltpu.sync_copy(src_hbm.at[wid], src_v)
        pltpu.sync_copy(dst_hbm.at[wid], dst_v)

        def issue_gather(g, b):
            base8 = wid * (EPT // 8) + g * 16
            pltpu.async_copy(xw_hbm.at[src_v.at[g]], gx[b], sems[6 * b + 0])
            pltpu.async_copy(tabs_hbm.at[src_v.at[g]], gs[b], sems[6 * b + 1])
            pltpu.async_copy(tabd_hbm.at[dst_v.at[g]], gad[b], sems[6 * b + 2])
            pltpu.async_copy(ale_hbm.at[pl.ds(base8, 16)], gae[b], sems[6 * b + 3])

        def wait_gather(g, b):
            base8 = wid * (EPT // 8) + g * 16
            pltpu.make_async_copy(xw_hbm.at[src_v.at[g]], gx[b], sems[6 * b + 0]).wait()
            pltpu.make_async_copy(tabs_hbm.at[src_v.at[g]], gs[b], sems[6 * b + 1]).wait()
            pltpu.make_async_copy(tabd_hbm.at[dst_v.at[g]], gad[b], sems[6 * b + 2]).wait()
            pltpu.make_async_copy(ale_hbm.at[pl.ds(base8, 16)], gae[b], sems[6 * b + 3]).wait()

        def compute(b):
            gxb, gsb, gadb, gaeb, exbb = gx[b], gs[b], gad[b], gae[b], exb[b]

            @plsc.parallel_loop(0, 16, unroll=2)
            def row(r):
                for j in range(8):
                    e = 8 * r + j
                    a = gsb[e, :] + gadb[e, :] + gaeb[r, 16 * j:16 * j + 16]
                    a = jnp.where(a > 0.0, a, a * 0.2)
                    exv = jnp.exp(a)
                    exbb[e, :] = exv
                    for h in range(HEADS):
                        bv = jnp.full((16,), exv[h], jnp.float32)
                        gxb[e, 32 * h:32 * h + 16] = gxb[e, 32 * h:32 * h + 16] * bv
                        gxb[e, 32 * h + 16:32 * h + 32] = gxb[e, 32 * h + 16:32 * h + 32] * bv

        def issue_scatter(g, b):
            pltpu.async_copy(gx[b], accm.at[dst_v.at[g]], sems[6 * b + 4], add=True)
            pltpu.async_copy(exb[b], accd.at[dst_v.at[g]], sems[6 * b + 5], add=True)

        def wait_scatter(g, b):
            pltpu.make_async_copy(gx[b], accm.at[dst_v.at[g]], sems[6 * b + 4]).wait()
            pltpu.make_async_copy(exb[b], accd.at[dst_v.at[g]], sems[6 * b + 5]).wait()

        npairs = GRP // 2
        issue_gather(0, 0)

        def pair(p, c):
            g0 = 2 * p
            g1 = 2 * p + 1

            @pl.when(p > 0)
            def _():
                wait_scatter(g1, 1)

            issue_gather(g1, 1)
            wait_gather(g0, 0)
            compute(0)
            issue_scatter(g0, 0)
            wait_gather(g1, 1)
            compute(1)
            issue_scatter(g1, 1)
            wait_scatter(g0, 0)

            @pl.when(p < npairs - 1)
            def _():
                issue_gather(g0 + 2, 0)

            return c

        lax.fori_loop(0, npairs, pair, 0)
        wait_scatter(GRP - 1, 1)
        plsc.subcore_barrier()
        for rep in range(rps // 128):
            r0 = sid * rps + rep * 128
            pltpu.sync_copy(accm.at[pl.ds(r0, 128)], omsg_hbm.at[cid, pl.ds(r0, 128)])
            pltpu.sync_copy(accd.at[pl.ds(r0, 128)], oden_hbm.at[cid, pl.ds(r0, 128)])

    return edge_call


_edge_call = _make_edge_call()


def _dense0_body(x_ref, fcw_ref, fcb_ref, w_ref, asd_ref, ps_ref, pd_ref,
                 xw_ref, tabs_ref, tabd_ref):
    h0 = x_ref[...] @ fcw_ref[...] + fcb_ref[...]
    xw = h0 @ w_ref[...]
    sal = xw @ asd_ref[...]
    xw_ref[...] = xw
    tabs_ref[...] = sal @ ps_ref[...]
    tabd_ref[...] = sal @ pd_ref[...]


def _dense0(x_cat, fcw, fcb, w, asd, ps, pd):
    return pl.pallas_call(
        _dense0_body,
        grid=(NB,),
        in_specs=[
            pl.BlockSpec((BLK, NF + EMB), lambda i: (i, 0)),
            pl.BlockSpec((NF + EMB, 3 * EMB), lambda i: (0, 0)),
            pl.BlockSpec((1, 3 * EMB), lambda i: (0, 0)),
            pl.BlockSpec((3 * EMB, D), lambda i: (0, 0)),
            pl.BlockSpec((D, 2 * HEADS), lambda i: (0, 0)),
            pl.BlockSpec((2 * HEADS, 16), lambda i: (0, 0)),
            pl.BlockSpec((2 * HEADS, 16), lambda i: (0, 0)),
        ],
        out_specs=[
            pl.BlockSpec((BLK, D), lambda i: (i, 0)),
            pl.BlockSpec((BLK, 16), lambda i: (i, 0)),
            pl.BlockSpec((BLK, 16), lambda i: (i, 0)),
        ],
        out_shape=[
            jax.ShapeDtypeStruct((N, D), jnp.float32),
            jax.ShapeDtypeStruct((N, 16), jnp.float32),
            jax.ShapeDtypeStruct((N, 16), jnp.float32),
        ],
    )(x_cat, fcw, fcb, w, asd, ps, pd)


def _mid_body(pm_ref, pd_ref, b_ref, qm_ref, w_ref, asd_ref, ps_ref, psd_ref,
              xw_ref, tabs_ref, tabd_ref):
    den = (pd_ref[0] + pd_ref[1]) @ qm_ref[...]
    h = (pm_ref[0] + pm_ref[1]) / (den + 1e-16) + b_ref[...]
    h = jnp.where(h > 0.0, h, 0.01 * h)
    xw = h @ w_ref[...]
    sal = xw @ asd_ref[...]
    xw_ref[...] = xw
    tabs_ref[...] = sal @ ps_ref[...]
    tabd_ref[...] = sal @ psd_ref[...]


def _mid(pmsg, pden, bias, qmat, w, asd, ps, pd):
    return pl.pallas_call(
        _mid_body,
        grid=(NB,),
        in_specs=[
            pl.BlockSpec((2, BLK, D), lambda i: (0, i, 0)),
            pl.BlockSpec((2, BLK, 16), lambda i: (0, i, 0)),
            pl.BlockSpec((1, D), lambda i: (0, 0)),
            pl.BlockSpec((16, D), lambda i: (0, 0)),
            pl.BlockSpec((D, D), lambda i: (0, 0)),
            pl.BlockSpec((D, 2 * HEADS), lambda i: (0, 0)),
            pl.BlockSpec((2 * HEADS, 16), lambda i: (0, 0)),
            pl.BlockSpec((2 * HEADS, 16), lambda i: (0, 0)),
        ],
        out_specs=[
            pl.BlockSpec((BLK, D), lambda i: (i, 0)),
            pl.BlockSpec((BLK, 16), lambda i: (i, 0)),
            pl.BlockSpec((BLK, 16), lambda i: (i, 0)),
        ],
        out_shape=[
            jax.ShapeDtypeStruct((N, D), jnp.float32),
            jax.ShapeDtypeStruct((N, 16), jnp.float32),
            jax.ShapeDtypeStruct((N, 16), jnp.float32),
        ],
    )(pmsg, pden, bias, qmat, w, asd, ps, pd)


def _post_body(pm_ref, pd_ref, b_ref, qm_ref, wq_ref, bq_ref, wk_ref, bk_ref,
               wv_ref, bv_ref, q_ref, k_ref, v_ref):
    den = (pd_ref[0] + pd_ref[1]) @ qm_ref[...]
    h = (pm_ref[0] + pm_ref[1]) / (den + 1e-16) + b_ref[...]
    h = jnp.where(h > 0.0, h, 0.01 * h)
    q_ref[...] = h @ wq_ref[...] + bq_ref[...]
    k_ref[...] = h @ wk_ref[...] + bk_ref[...]
    v_ref[...] = h @ wv_ref[...] + bv_ref[...]


def _post(pmsg, pden, bias, qmat, wq, bq, wk, bk, wv, bv):
    wspec = pl.BlockSpec((D, D), lambda i: (0, 0))
    bspec = pl.BlockSpec((1, D), lambda i: (0, 0))
    nspec = pl.BlockSpec((BLK, D), lambda i: (i, 0))
    return pl.pallas_call(
        _post_body,
        grid=(NB,),
        in_specs=[
            pl.BlockSpec((2, BLK, D), lambda i: (0, i, 0)),
            pl.BlockSpec((2, BLK, 16), lambda i: (0, i, 0)),
            bspec,
            pl.BlockSpec((16, D), lambda i: (0, 0)),
            wspec, bspec, wspec, bspec, wspec, bspec,
        ],
        out_specs=[nspec, nspec, nspec],
        out_shape=[jax.ShapeDtypeStruct((N, D), jnp.float32)] * 3,
    )(pmsg, pden, bias, qmat, wq, bq, wk, bk, wv, bv)


def _ale_body(ea_ref, bd_ref, *outs):
    ea = ea_ref[...]
    for l in range(NGAT):
        outs[l][...] = ea @ bd_ref[...][:, 128 * l:128 * (l + 1)]


def _ale(ea_pack, bd_all):
    eb = E // 8 // 32
    return pl.pallas_call(
        _ale_body,
        grid=(32,),
        in_specs=[
            pl.BlockSpec((eb, 128), lambda i: (i, 0)),
            pl.BlockSpec((128, 128 * NGAT), lambda i: (0, 0)),
        ],
        out_specs=[pl.BlockSpec((eb, 128), lambda i: (i, 0))] * NGAT,
        out_shape=[jax.ShapeDtypeStruct((E // 8, 128), jnp.float32)] * NGAT,
    )(ea_pack, bd_all)


QB = 256
_SCALE = 1.0 / np.sqrt(GC)


def _attn_body(q_ref, k_ref, v_ref, wop_ref, cbp_ref, z_ref):
    bf16 = jnp.bfloat16
    q = (q_ref[0] * _SCALE).astype(bf16)
    k = k_ref[0].astype(bf16)
    v = v_ref[0].astype(bf16)
    outs = []
    for h in range(HEADS):
        sl = slice(GC * h, GC * (h + 1))
        s = lax.dot_general(q[:, sl], k[:, sl],
                            (((1,), (1,)), ((), ())),
                            preferred_element_type=jnp.float32)
        p = jnp.exp(s)
        rs = jnp.sum(p, axis=1, keepdims=True)
        oh = lax.dot_general(p.astype(bf16), v[:, sl],
                             (((1,), (0,)), ((), ())),
                             preferred_element_type=jnp.float32)
        outs.append(oh / rs)
    o = jnp.concatenate(outs, axis=1)
    z_ref[0] = o @ wop_ref[...] + cbp_ref[...]


def _attn(q, k, v, wop, cbp):
    return pl.pallas_call(
        _attn_body,
        grid=(BATCH, NUM_NODES // QB),
        in_specs=[
            pl.BlockSpec((1, QB, D), lambda b, i: (b, i, 0)),
            pl.BlockSpec((1, NUM_NODES, D), lambda b, i: (b, 0, 0)),
            pl.BlockSpec((1, NUM_NODES, D), lambda b, i: (b, 0, 0)),
            pl.BlockSpec((D, PROJ), lambda b, i: (0, 0)),
            pl.BlockSpec((1, PROJ), lambda b, i: (0, 0)),
        ],
        out_specs=pl.BlockSpec((1, QB, PROJ), lambda b, i: (b, i, 0)),
        out_shape=jax.ShapeDtypeStruct((BATCH, NUM_NODES, PROJ), jnp.float32),
    )(q, k, v, wop, cbp)


def _fin_body(z_ref, w_ref, b_ref, o_ref):
    @pl.when(pl.program_id(0) == 0)
    def _():
        o_ref[...] = jnp.broadcast_to(b_ref[...], (BATCH, OUT))

    o_ref[...] += z_ref[...] @ w_ref[...]


def _fin(z2, out_w, out_b):
    kb = 1024
    return pl.pallas_call(
        _fin_body,
        grid=(NUM_NODES * PROJ // kb),
        in_specs=[
            pl.BlockSpec((BATCH, kb), lambda i: (0, i)),
            pl.BlockSpec((kb, OUT), lambda i: (i, 0)),
            pl.BlockSpec((1, OUT), lambda i: (0, 0)),
        ],
        out_specs=pl.BlockSpec((BATCH, OUT), lambda i: (0, 0)),
        out_shape=jax.ShapeDtypeStruct((BATCH, OUT), jnp.float32),
    )(z2, out_w, out_b)


def _make_asd(a_s, a_d):
    hidx = jnp.arange(D) // GC
    sel = (hidx[:, None] == jnp.arange(HEADS)[None, :]).astype(jnp.float32)
    return jnp.concatenate(
        [a_s.reshape(D)[:, None] * sel, a_d.reshape(D)[:, None] * sel], axis=1)


def _make_be(we, ae):
    return jnp.pad(
        jnp.einsum("dhc,hc->dh", we.reshape(ED, HEADS, GC), ae.reshape(HEADS, GC)),
        ((0, 0), (0, 16 - HEADS)))


def kernel(x, edge_index, edge_attr, batch, emb, fc_W, fc_b, g0_W, g0_as,
           g0_ad, g0_We, g0_ae, g0_b, gW, gAs, gAd, gWe, gAe, gB, Wq, bq,
           Wk, bk, Wv, bv, Wo, bo, proj_W, proj_b, out_W, out_b):
    f32 = jnp.float32
    x_cat = jnp.concatenate([x, jnp.tile(emb, (BATCH, 1))], axis=1)

    ps = (jnp.arange(2 * HEADS)[:, None] ==
          jnp.arange(16)[None, :]).astype(f32)
    pd = (jnp.arange(2 * HEADS)[:, None] ==
          (jnp.arange(16)[None, :] + HEADS)).astype(f32)
    qmat = (jnp.arange(16)[:, None] ==
            (jnp.arange(D)[None, :] // GC)).astype(f32)

    asd = [_make_asd(g0_as, g0_ad)] + [
        _make_asd(gAs[i], gAd[i]) for i in range(NGAT - 1)]
    eye8 = jnp.eye(8, dtype=f32)
    bd_all = jnp.concatenate(
        [jnp.kron(eye8, _make_be(g0_We, g0_ae))] +
        [jnp.kron(eye8, _make_be(gWe[i], gAe[i])) for i in range(NGAT - 1)],
        axis=1)
    ws = [g0_W] + [gW[i] for i in range(NGAT - 1)]
    biases = [g0_b] + [gB[i] for i in range(NGAT - 1)]

    ale = _ale(edge_attr.reshape(E // 8, 8 * ED), bd_all)
    src3 = edge_index[0].reshape(NW, GRP, 128)
    dst3 = edge_index[1].reshape(NW, GRP, 128)

    xw, tabs, tabd = _dense0(x_cat, fc_W, fc_b.reshape(1, -1), ws[0],
                             asd[0], ps, pd)
    for l in range(NGAT):
        pmsg, pden = _edge_call(xw, tabs, tabd, ale[l], src3, dst3)
        if l < NGAT - 1:
            xw, tabs, tabd = _mid(pmsg, pden, biases[l].reshape(1, -1), qmat,
                                  ws[l + 1], asd[l + 1], ps, pd)
        else:
            q, k, v = _post(pmsg, pden, biases[l].reshape(1, -1), qmat,
                            Wq, bq.reshape(1, -1), Wk, bk.reshape(1, -1),
                            Wv, bv.reshape(1, -1))

    wop = Wo @ proj_W
    cbp = (bo @ proj_W + proj_b).reshape(1, -1)
    z = _attn(q.reshape(BATCH, NUM_NODES, D),
              k.reshape(BATCH, NUM_NODES, D),
              v.reshape(BATCH, NUM_NODES, D), wop, cbp)
    z2 = z.reshape(BATCH, NUM_NODES * PROJ)
    return _fin(z2, out_W, out_b.reshape(1, -1))
